# baseline (speedup 1.0000x reference)
import functools

import jax
import jax.numpy as jnp
from jax import lax
from jax.experimental import pallas as pl
from jax.experimental.pallas import tpu as pltpu
from jax.experimental.pallas import tpu_sc as plsc

_B = 16384
_D_UP = 64
_D_SM = 32
_D_MODEL = 512

_NC = 2
_NS = 16
_NW = _NC * _NS
_BPW = _B // _NW
_CHUNK = 128
_NCHUNK = _BPW // _CHUNK


def _sc_gather_body(up_idx, pf_idx, dt_idx, lk_idx,
                    up_tab, pf_tab, dt_tab, lk_tab,
                    e_up, e_pf, e_dt, e_lk,
                    i_up, i_pf, i_dt, i_lk,
                    r_up, r_pf, r_dt, r_lk, sem):
    wid = lax.axis_index("s") * _NC + lax.axis_index("c")
    base = wid * _BPW
    crow = wid * _NCHUNK

    pltpu.sync_copy(up_idx.at[pl.ds(crow, _NCHUNK)], i_up)
    pltpu.sync_copy(pf_idx.at[pl.ds(crow, _NCHUNK)], i_pf)
    pltpu.sync_copy(dt_idx.at[pl.ds(crow, _NCHUNK)], i_dt)
    pltpu.sync_copy(lk_idx.at[pl.ds(crow, _NCHUNK)], i_lk)

    copies = []
    for j in range(_NCHUNK):
        o = j * _CHUNK
        copies.append(pltpu.async_copy(
            up_tab.at[i_up.at[j]], r_up.at[pl.ds(o, _CHUNK)], sem))
        copies.append(pltpu.async_copy(
            pf_tab.at[i_pf.at[j]], r_pf.at[pl.ds(o, _CHUNK)], sem))
        copies.append(pltpu.async_copy(
            dt_tab.at[i_dt.at[j]], r_dt.at[pl.ds(o, _CHUNK)], sem))
        copies.append(pltpu.async_copy(
            lk_tab.at[i_lk.at[j]], r_lk.at[pl.ds(o, _CHUNK)], sem))
    for c in copies:
        c.wait()

    pltpu.sync_copy(r_up, e_up.at[pl.ds(base, _BPW)])
    pltpu.sync_copy(r_pf, e_pf.at[pl.ds(base, _BPW)])
    pltpu.sync_copy(r_dt, e_dt.at[pl.ds(base, _BPW)])
    pltpu.sync_copy(r_lk, e_lk.at[pl.ds(base, _BPW)])


@functools.cache
def _sc_gather():
    return pl.kernel(
        _sc_gather_body,
        out_type=[
            jax.ShapeDtypeStruct((_B, _D_UP), jnp.float32),
            jax.ShapeDtypeStruct((_B, _D_SM), jnp.float32),
            jax.ShapeDtypeStruct((_B, _D_SM), jnp.float32),
            jax.ShapeDtypeStruct((_B, _D_SM), jnp.float32),
        ],
        mesh=plsc.VectorSubcoreMesh(core_axis_name="c", subcore_axis_name="s",
                                    num_cores=_NC, num_subcores=_NS),
        scratch_types=[
            pltpu.VMEM((_NCHUNK, _CHUNK), jnp.int32),
            pltpu.VMEM((_NCHUNK, _CHUNK), jnp.int32),
            pltpu.VMEM((_NCHUNK, _CHUNK), jnp.int32),
            pltpu.VMEM((_NCHUNK, _CHUNK), jnp.int32),
            pltpu.VMEM((_BPW, _D_UP), jnp.float32),
            pltpu.VMEM((_BPW, _D_SM), jnp.float32),
            pltpu.VMEM((_BPW, _D_SM), jnp.float32),
            pltpu.VMEM((_BPW, _D_SM), jnp.float32),
            pltpu.SemaphoreType.DMA,
        ],
        compiler_params=pltpu.CompilerParams(use_tc_tiling_on_sc=False),
    )


_BLK = 2048


def _mlp_body(e_up, e_pf, e_dt, e_lk, W1, b1, gamma, beta, W2, b2, out):
    h = jnp.dot(e_up[...], W1[0:64, :], preferred_element_type=jnp.float32)
    h = h + jnp.dot(e_pf[...], W1[64:96, :], preferred_element_type=jnp.float32)
    h = h + jnp.dot(e_dt[...], W1[96:128, :], preferred_element_type=jnp.float32)
    h = h + jnp.dot(e_lk[...], W1[128:160, :], preferred_element_type=jnp.float32)
    h = h + b1[...]
    mean = jnp.mean(h, axis=-1, keepdims=True)
    c = h - mean
    var = jnp.mean(c * c, axis=-1, keepdims=True)
    h = c * lax.rsqrt(var + 1e-5) * gamma[...] + beta[...]
    h = jnp.maximum(h, 0.0)
    out[...] = jnp.dot(h, W2[...], preferred_element_type=jnp.float32) + b2[...]


def _mlp(e_up, e_pf, e_dt, e_lk, W1, b1, gamma, beta, W2, b2):
    grid = (_B // _BLK,)
    return pl.pallas_call(
        _mlp_body,
        grid=grid,
        in_specs=[
            pl.BlockSpec((_BLK, _D_UP), lambda i: (i, 0)),
            pl.BlockSpec((_BLK, _D_SM), lambda i: (i, 0)),
            pl.BlockSpec((_BLK, _D_SM), lambda i: (i, 0)),
            pl.BlockSpec((_BLK, _D_SM), lambda i: (i, 0)),
            pl.BlockSpec((160, _D_MODEL), lambda i: (0, 0)),
            pl.BlockSpec((1, _D_MODEL), lambda i: (0, 0)),
            pl.BlockSpec((1, _D_MODEL), lambda i: (0, 0)),
            pl.BlockSpec((1, _D_MODEL), lambda i: (0, 0)),
            pl.BlockSpec((_D_MODEL, _D_MODEL), lambda i: (0, 0)),
            pl.BlockSpec((1, _D_MODEL), lambda i: (0, 0)),
        ],
        out_specs=pl.BlockSpec((_BLK, _D_MODEL), lambda i: (i, 0)),
        out_shape=jax.ShapeDtypeStruct((_B, _D_MODEL), jnp.float32),
        compiler_params=pltpu.CompilerParams(
            dimension_semantics=("arbitrary",),
        ),
    )(e_up, e_pf, e_dt, e_lk, W1, b1, gamma, beta, W2, b2)


def kernel(uploader, platform, date, likes, uploader_table, platform_table,
           date_table, likes_table, W1, b1, gamma, beta, W2, b2):
    up_idx = uploader.astype(jnp.int32).reshape(_B // _CHUNK, _CHUNK)
    pf_idx = platform.astype(jnp.int32).reshape(_B // _CHUNK, _CHUNK)
    dt_idx = date.astype(jnp.int32).reshape(_B // _CHUNK, _CHUNK)
    lk_idx = likes.astype(jnp.int32).reshape(_B // _CHUNK, _CHUNK)

    e_up, e_pf, e_dt, e_lk = _sc_gather()(
        up_idx, pf_idx, dt_idx, lk_idx,
        uploader_table, platform_table, date_table, likes_table)

    out = _mlp(e_up, e_pf, e_dt, e_lk,
               W1, b1.reshape(1, _D_MODEL), gamma.reshape(1, _D_MODEL),
               beta.reshape(1, _D_MODEL), W2, b2.reshape(1, _D_MODEL))
    return out[:, None, :]
